# baseline (device time: 28748 ns/iter reference)
import jax
import jax.numpy as jnp
from jax import lax
from jax.experimental import pallas as pl
from jax.experimental.pallas import tpu as pltpu

B = 4
S = 512
S_OUT = 256
SQ = 128
K = 512
N = 1024
CPB = 2
CR = SQ // CPB
NC = B * CPB


def kernel(O, Wo):
    my_x = lax.axis_index("x")
    my_y = lax.axis_index("y")
    ix = my_x * 2 + my_y

    def mk(xv, yv):
        def f(Ofull):
            myq = xv * S_OUT + yv * SQ
            nbq = (1 - xv) * S_OUT + yv * SQ
            On = Ofull[:, nbq:nbq + SQ].reshape(B, SQ, K).astype(jnp.bfloat16)
            Om = Ofull[:, myq:myq + SQ].reshape(B, SQ, K).astype(jnp.bfloat16)
            return On, Om
        return f

    O_nb, O_my = lax.switch(ix, [mk(0, 0), mk(0, 1), mk(1, 0), mk(1, 1)], O)
    Wo_bf = Wo.astype(jnp.bfloat16)

    def body(o_nb_ref, o_my_ref, w_ref, out_ref, acc, oth,
             xsend_buf, xrecv_buf, ystage, yrecv_buf,
             xsend_sems, xrecv_sems, ysend_sems, yrecv_sems,
             own_out_sems, oth_out_sems):
        my_x = lax.axis_index("x")
        my_y = lax.axis_index("y")
        ox = 1 - my_x
        oy = 1 - my_y

        barrier = pltpu.get_barrier_semaphore()
        pl.semaphore_signal(
            barrier, inc=1,
            device_id=(ox, my_y), device_id_type=pl.DeviceIdType.MESH,
        )
        pl.semaphore_signal(
            barrier, inc=1,
            device_id=(my_x, oy), device_id_type=pl.DeviceIdType.MESH,
        )

        loc = my_y * SQ
        sloc = oy * SQ

        x_rdmas = []
        for c in range(NC):
            b, half = divmod(c, CPB)
            xsend_buf[c] = jnp.dot(
                o_nb_ref[b, pl.ds(half * CR, CR), :],
                w_ref[...],
                preferred_element_type=jnp.float32,
            ).astype(jnp.bfloat16)
            if c == 0:
                pl.semaphore_wait(barrier, 2)
            rdma = pltpu.make_async_remote_copy(
                src_ref=xsend_buf.at[c],
                dst_ref=xrecv_buf.at[c],
                send_sem=xsend_sems.at[c],
                recv_sem=xrecv_sems.at[c],
                device_id=(ox, my_y),
                device_id_type=pl.DeviceIdType.MESH,
            )
            rdma.start()
            x_rdmas.append(rdma)

        for b in range(B):
            acc[b] = jnp.dot(
                o_my_ref[b],
                w_ref[...],
                preferred_element_type=jnp.float32,
            )

        y_rdmas = []
        own_out = []
        for c in range(NC):
            b, half = divmod(c, CPB)
            x_rdmas[c].wait()
            r = acc[b, pl.ds(half * CR, CR), :] + xrecv_buf[c].astype(jnp.float32)
            acc[b, pl.ds(half * CR, CR), :] = r
            ystage[c] = r.astype(jnp.bfloat16)
            yr = pltpu.make_async_remote_copy(
                src_ref=ystage.at[c],
                dst_ref=yrecv_buf.at[c],
                send_sem=ysend_sems.at[c],
                recv_sem=yrecv_sems.at[c],
                device_id=(my_x, oy),
                device_id_type=pl.DeviceIdType.MESH,
            )
            yr.start()
            y_rdmas.append(yr)
            cp = pltpu.make_async_copy(
                acc.at[b, pl.ds(half * CR, CR), :],
                out_ref.at[b, pl.ds(loc + half * CR, CR), :],
                own_out_sems.at[c],
            )
            cp.start()
            own_out.append(cp)

        oth_out = []
        for c in range(NC):
            b, half = divmod(c, CPB)
            y_rdmas[c].wait()
            oth[c] = yrecv_buf[c].astype(jnp.float32)
            cp = pltpu.make_async_copy(
                oth.at[c],
                out_ref.at[b, pl.ds(sloc + half * CR, CR), :],
                oth_out_sems.at[c],
            )
            cp.start()
            oth_out.append(cp)

        for c in range(NC):
            own_out[c].wait()
            oth_out[c].wait()

    return pl.pallas_call(
        body,
        out_shape=jax.ShapeDtypeStruct((B, S_OUT, N), jnp.float32),
        in_specs=[
            pl.BlockSpec(memory_space=pltpu.VMEM),
            pl.BlockSpec(memory_space=pltpu.VMEM),
            pl.BlockSpec(memory_space=pltpu.VMEM),
        ],
        out_specs=pl.BlockSpec(memory_space=pl.ANY),
        scratch_shapes=[
            pltpu.VMEM((B, SQ, N), jnp.float32),
            pltpu.VMEM((NC, CR, N), jnp.float32),
            pltpu.VMEM((NC, CR, N), jnp.bfloat16),
            pltpu.VMEM((NC, CR, N), jnp.bfloat16),
            pltpu.VMEM((NC, CR, N), jnp.bfloat16),
            pltpu.VMEM((NC, CR, N), jnp.bfloat16),
            pltpu.SemaphoreType.DMA((NC,)),
            pltpu.SemaphoreType.DMA((NC,)),
            pltpu.SemaphoreType.DMA((NC,)),
            pltpu.SemaphoreType.DMA((NC,)),
            pltpu.SemaphoreType.DMA((NC,)),
            pltpu.SemaphoreType.DMA((NC,)),
        ],
        compiler_params=pltpu.CompilerParams(collective_id=0),
    )(O_nb, O_my, Wo_bf)
